# baseline (speedup 1.0000x reference)
import functools

import jax
import jax.numpy as jnp
from jax.experimental import pallas as pl
from jax.experimental.pallas import tpu as pltpu

N = 10000
E = 160000
D = 256
NPAD = 10240
RB = 1024


def _matvec_body(h_ref, w_ref, o_ref):
    o_ref[...] = jnp.dot(h_ref[...], w_ref[...], preferred_element_type=jnp.float32)


def _matvec(h, W):
    return pl.pallas_call(
        _matvec_body,
        grid=(10,),
        in_specs=[pl.BlockSpec((1000, D), lambda i: (i, 0)),
                  pl.BlockSpec((D, 1), lambda i: (0, 0))],
        out_specs=pl.BlockSpec((1000, 1), lambda i: (i, 0)),
        out_shape=jax.ShapeDtypeStruct((N, 1), jnp.float32),
    )(h, W)


def _rank_body(vi_ref, vj_ref, o_ref):
    i = pl.program_id(0)
    j = pl.program_id(1)
    a = vi_ref[...]
    b = vj_ref[...]
    A = a[:, None]
    B = b[None, :]
    gi = i * RB + jax.lax.broadcasted_iota(jnp.int32, (RB, RB), 0)
    gj = j * RB + jax.lax.broadcasted_iota(jnp.int32, (RB, RB), 1)
    less = (B < A) | ((B == A) & (gj < gi))
    cnt = jnp.sum(less.astype(jnp.int32), axis=1)

    @pl.when(j == 0)
    def _init():
        o_ref[...] = cnt

    @pl.when(j != 0)
    def _acc():
        o_ref[...] = o_ref[...] + cnt


def _ranks(v):
    return pl.pallas_call(
        _rank_body,
        grid=(NPAD // RB, NPAD // RB),
        in_specs=[pl.BlockSpec((RB,), lambda i, j: (i,)),
                  pl.BlockSpec((RB,), lambda i, j: (j,))],
        out_specs=pl.BlockSpec((RB,), lambda i, j: (i,)),
        out_shape=jax.ShapeDtypeStruct((NPAD,), jnp.int32),
        compiler_params=pltpu.CompilerParams(
            dimension_semantics=("arbitrary", "arbitrary")),
    )(v, v)


def _scatter_body(rank_ref, scores_ref, h_ref, wt_ref, out_ref, perm_ref):
    i = pl.program_id(0)
    s = scores_ref[i]
    out_ref[...] = h_ref[...] + s * wt_ref[...]
    perm_ref[...] = jnp.full((1, 1, 1), i, jnp.int32)


def _reorder(h, scores, w_projT, rank):
    grid_spec = pltpu.PrefetchScalarGridSpec(
        num_scalar_prefetch=2,
        grid=(N,),
        in_specs=[
            pl.BlockSpec((1, 1, D), lambda i, rank_ref, scores_ref: (i, 0, 0)),
            pl.BlockSpec((1, 1, D), lambda i, rank_ref, scores_ref: (0, 0, 0)),
        ],
        out_specs=[
            pl.BlockSpec((1, 1, D), lambda i, rank_ref, scores_ref: (rank_ref[i], 0, 0)),
            pl.BlockSpec((1, 1, 1), lambda i, rank_ref, scores_ref: (rank_ref[i], 0, 0)),
        ],
    )
    h_ordered, perm3 = pl.pallas_call(
        _scatter_body,
        grid_spec=grid_spec,
        out_shape=[jax.ShapeDtypeStruct((N, 1, D), jnp.float32),
                   jax.ShapeDtypeStruct((N, 1, 1), jnp.int32)],
    )(rank, scores, h.reshape(N, 1, D), w_projT.reshape(1, 1, D))
    return h_ordered.reshape(N, D), perm3.reshape(N)


def kernel(h, edge_index, W, att_src, att_dst, bias, w_proj):
    x = _matvec(h, W)

    a_s = (x * att_src).sum(-1)
    a_d = (x * att_dst).sum(-1)
    loop = jnp.arange(N, dtype=edge_index.dtype)
    src = jnp.concatenate([edge_index[0], loop])
    dst = jnp.concatenate([edge_index[1], loop])
    e = a_s[src] + a_d[dst]
    e = jnp.where(e > 0, e, 0.2 * e)
    m = jax.ops.segment_max(e, dst, num_segments=N)
    ex = jnp.exp(e - m[dst])
    den = jax.ops.segment_sum(ex, dst, num_segments=N)
    alpha = ex / den[dst]
    out = jax.ops.segment_sum(alpha[:, None] * x[src], dst, num_segments=N) + bias
    scores = out[:, 0]

    c = -scores
    c = jnp.where(c == 0.0, jnp.float32(0.0), c)
    b = jax.lax.bitcast_convert_type(c, jnp.int32)
    v = jnp.where(b >= 0, b, (~b) ^ jnp.int32(-2147483648))
    vpad = jnp.concatenate([v, jnp.full((NPAD - N,), jnp.int32(2147483647))])
    rank = _ranks(vpad)[:N]

    h_ordered, perm_idx = _reorder(h, scores, w_proj.T, rank)
    return (h_ordered, perm_idx, scores)


# speedup vs baseline: 1.7869x; 1.7869x over previous
import functools

import jax
import jax.numpy as jnp
from jax import lax
from jax.experimental import pallas as pl
from jax.experimental.pallas import tpu as pltpu
from jax.experimental.pallas import tpu_sc as plsc

N = 10000
E = 160000
D = 256
NPAD = 10240
RB = 1024


def _matvec_body(h_ref, w_ref, o_ref):
    o_ref[...] = jnp.dot(h_ref[...], w_ref[...], preferred_element_type=jnp.float32)


def _matvec(h, W):
    return pl.pallas_call(
        _matvec_body,
        grid=(10,),
        in_specs=[pl.BlockSpec((1000, D), lambda i: (i, 0)),
                  pl.BlockSpec((D, 1), lambda i: (0, 0))],
        out_specs=pl.BlockSpec((1000, 1), lambda i: (i, 0)),
        out_shape=jax.ShapeDtypeStruct((N, 1), jnp.float32),
    )(h, W)


def _rank_body(vi_ref, vj_ref, o_ref):
    i = pl.program_id(0)
    j = pl.program_id(1)
    a = vi_ref[...]
    b = vj_ref[...]
    A = a[:, None]
    B = b[None, :]
    gi = i * RB + jax.lax.broadcasted_iota(jnp.int32, (RB, RB), 0)
    gj = j * RB + jax.lax.broadcasted_iota(jnp.int32, (RB, RB), 1)
    less = (B < A) | ((B == A) & (gj < gi))
    cnt = jnp.sum(less.astype(jnp.int32), axis=1)

    @pl.when(j == 0)
    def _init():
        o_ref[...] = cnt

    @pl.when(j != 0)
    def _acc():
        o_ref[...] = o_ref[...] + cnt


def _ranks(v):
    return pl.pallas_call(
        _rank_body,
        grid=(NPAD // RB, NPAD // RB),
        in_specs=[pl.BlockSpec((RB,), lambda i, j: (i,)),
                  pl.BlockSpec((RB,), lambda i, j: (j,))],
        out_specs=pl.BlockSpec((RB,), lambda i, j: (i,)),
        out_shape=jax.ShapeDtypeStruct((NPAD,), jnp.int32),
        compiler_params=pltpu.CompilerParams(
            dimension_semantics=("arbitrary", "arbitrary")),
    )(v, v)


def _scatter_body(rank_ref, scores_ref, h_ref, wt_ref, out_ref, perm_ref):
    i = pl.program_id(0)
    s = scores_ref[i]
    out_ref[...] = h_ref[...] + s * wt_ref[...]
    perm_ref[...] = jnp.full((1, 1, 1), i, jnp.int32)


def _reorder(h, scores, w_projT, rank):
    grid_spec = pltpu.PrefetchScalarGridSpec(
        num_scalar_prefetch=2,
        grid=(N,),
        in_specs=[
            pl.BlockSpec((1, 1, D), lambda i, rank_ref, scores_ref: (i, 0, 0)),
            pl.BlockSpec((1, 1, D), lambda i, rank_ref, scores_ref: (0, 0, 0)),
        ],
        out_specs=[
            pl.BlockSpec((1, 1, D), lambda i, rank_ref, scores_ref: (rank_ref[i], 0, 0)),
            pl.BlockSpec((1, 1, 1), lambda i, rank_ref, scores_ref: (rank_ref[i], 0, 0)),
        ],
    )
    h_ordered, perm3 = pl.pallas_call(
        _scatter_body,
        grid_spec=grid_spec,
        out_shape=[jax.ShapeDtypeStruct((N, 1, D), jnp.float32),
                   jax.ShapeDtypeStruct((N, 1, 1), jnp.int32)],
    )(rank, scores, h.reshape(N, 1, D), w_projT.reshape(1, 1, D))
    return h_ordered.reshape(N, D), perm3.reshape(N)


NW = 32
CH = NPAD // NW
SCB = 64
NSB = CH // SCB


def _sc_reorder_body(h_hbm, sc_hbm, rank3_hbm, io_hbm, wp_hbm,
                     out_hbm, perm_hbm,
                     rows_v, rank2_v, io_v, sc_v, wp_v, sem):
    wid = lax.axis_index("s") * 2 + lax.axis_index("c")
    base = wid * CH
    pltpu.sync_copy(h_hbm.at[pl.ds(base, CH)], rows_v)
    pltpu.sync_copy(rank3_hbm.at[wid], rank2_v)
    pltpu.sync_copy(io_hbm.at[wid], io_v)
    pltpu.sync_copy(sc_hbm.at[pl.ds(base, CH)], sc_v)
    pltpu.sync_copy(wp_hbm, wp_v)

    def group_body(k, _):
        s16 = sc_v[pl.ds(k * 16, 16)]
        for j in range(16):
            r = k * 16 + j
            s = s16[j]
            for dk in range(D // 16):
                sl = pl.ds(dk * 16, 16)
                rows_v[r, sl] = rows_v[r, sl] + s * wp_v[sl]
        return _

    lax.fori_loop(0, CH // 16, group_body, 0)

    for j in range(NSB):
        pltpu.async_copy(rows_v.at[pl.ds(j * SCB, SCB)],
                         out_hbm.at[rank2_v.at[j]], sem).wait()
    for j in range(NSB):
        pltpu.async_copy(io_v.at[j], perm_hbm.at[rank2_v.at[j]], sem).wait()


def _sc_reorder(h, scores, rank, w_proj1d):
    hp = jnp.concatenate([h, jnp.zeros((NPAD - N, D), jnp.float32)])
    sp = jnp.concatenate([scores, jnp.zeros((NPAD - N,), jnp.float32)])
    kfn = pl.kernel(
        _sc_reorder_body,
        mesh=plsc.VectorSubcoreMesh(core_axis_name="c", subcore_axis_name="s"),
        compiler_params=pltpu.CompilerParams(use_tc_tiling_on_sc=False),
        out_type=[jax.ShapeDtypeStruct((NPAD, D), jnp.float32),
                  jax.ShapeDtypeStruct((NPAD,), jnp.int32)],
        scratch_types=[
            pltpu.VMEM((CH, D), jnp.float32),
            pltpu.VMEM((NSB, SCB), jnp.int32),
            pltpu.VMEM((NSB, SCB), jnp.int32),
            pltpu.VMEM((CH,), jnp.float32),
            pltpu.VMEM((D,), jnp.float32),
            pltpu.SemaphoreType.DMA,
        ],
    )
    iov = jnp.arange(NPAD, dtype=jnp.int32).reshape(NW, NSB, SCB)
    out, perm = kfn(hp, sp, rank.reshape(NW, NSB, SCB), iov, w_proj1d)
    return out[:N], perm[:N]


def kernel(h, edge_index, W, att_src, att_dst, bias, w_proj):
    x = _matvec(h, W)

    a_s = (x * att_src).sum(-1)
    a_d = (x * att_dst).sum(-1)
    loop = jnp.arange(N, dtype=edge_index.dtype)
    src = jnp.concatenate([edge_index[0], loop])
    dst = jnp.concatenate([edge_index[1], loop])
    e = a_s[src] + a_d[dst]
    e = jnp.where(e > 0, e, 0.2 * e)
    m = jax.ops.segment_max(e, dst, num_segments=N)
    ex = jnp.exp(e - m[dst])
    den = jax.ops.segment_sum(ex, dst, num_segments=N)
    alpha = ex / den[dst]
    out = jax.ops.segment_sum(alpha[:, None] * x[src], dst, num_segments=N) + bias
    scores = out[:, 0]

    c = -scores
    c = jnp.where(c == 0.0, jnp.float32(0.0), c)
    b = jax.lax.bitcast_convert_type(c, jnp.int32)
    v = jnp.where(b >= 0, b, (~b) ^ jnp.int32(-2147483648))
    vpad = jnp.concatenate([v, jnp.full((NPAD - N,), jnp.int32(2147483647))])
    rank = _ranks(vpad)

    h_ordered, perm_idx = _sc_reorder(h, scores, rank, w_proj[:, 0])
    return (h_ordered, perm_idx, scores)


# speedup vs baseline: 1.8501x; 1.0353x over previous
import functools

import jax
import jax.numpy as jnp
from jax import lax
from jax.experimental import pallas as pl
from jax.experimental.pallas import tpu as pltpu
from jax.experimental.pallas import tpu_sc as plsc

N = 10000
E = 160000
D = 256
NPAD = 10240
RB = 1024


def _matvec_body(h_ref, w_ref, o_ref):
    o_ref[...] = jnp.dot(h_ref[...], w_ref[...], preferred_element_type=jnp.float32)


def _matvec(h, W):
    return pl.pallas_call(
        _matvec_body,
        grid=(10,),
        in_specs=[pl.BlockSpec((1000, D), lambda i: (i, 0)),
                  pl.BlockSpec((D, 1), lambda i: (0, 0))],
        out_specs=pl.BlockSpec((1000, 1), lambda i: (i, 0)),
        out_shape=jax.ShapeDtypeStruct((N, 1), jnp.float32),
    )(h, W)


def _rank_body(vi_ref, vj_ref, o_ref):
    i = pl.program_id(0)
    j = pl.program_id(1)
    a = vi_ref[...]
    b = vj_ref[...]
    A = a[:, None]
    B = b[None, :]
    gi = i * RB + jax.lax.broadcasted_iota(jnp.int32, (RB, RB), 0)
    gj = j * RB + jax.lax.broadcasted_iota(jnp.int32, (RB, RB), 1)
    less = (B < A) | ((B == A) & (gj < gi))
    cnt = jnp.sum(less.astype(jnp.int32), axis=1)

    @pl.when(j == 0)
    def _init():
        o_ref[...] = cnt

    @pl.when(j != 0)
    def _acc():
        o_ref[...] = o_ref[...] + cnt


def _ranks(v):
    return pl.pallas_call(
        _rank_body,
        grid=(NPAD // RB, NPAD // RB),
        in_specs=[pl.BlockSpec((RB,), lambda i, j: (i,)),
                  pl.BlockSpec((RB,), lambda i, j: (j,))],
        out_specs=pl.BlockSpec((RB,), lambda i, j: (i,)),
        out_shape=jax.ShapeDtypeStruct((NPAD,), jnp.int32),
        compiler_params=pltpu.CompilerParams(
            dimension_semantics=("arbitrary", "arbitrary")),
    )(v, v)


def _scatter_body(rank_ref, scores_ref, h_ref, wt_ref, out_ref, perm_ref):
    i = pl.program_id(0)
    s = scores_ref[i]
    out_ref[...] = h_ref[...] + s * wt_ref[...]
    perm_ref[...] = jnp.full((1, 1, 1), i, jnp.int32)


def _reorder(h, scores, w_projT, rank):
    grid_spec = pltpu.PrefetchScalarGridSpec(
        num_scalar_prefetch=2,
        grid=(N,),
        in_specs=[
            pl.BlockSpec((1, 1, D), lambda i, rank_ref, scores_ref: (i, 0, 0)),
            pl.BlockSpec((1, 1, D), lambda i, rank_ref, scores_ref: (0, 0, 0)),
        ],
        out_specs=[
            pl.BlockSpec((1, 1, D), lambda i, rank_ref, scores_ref: (rank_ref[i], 0, 0)),
            pl.BlockSpec((1, 1, 1), lambda i, rank_ref, scores_ref: (rank_ref[i], 0, 0)),
        ],
    )
    h_ordered, perm3 = pl.pallas_call(
        _scatter_body,
        grid_spec=grid_spec,
        out_shape=[jax.ShapeDtypeStruct((N, 1, D), jnp.float32),
                   jax.ShapeDtypeStruct((N, 1, 1), jnp.int32)],
    )(rank, scores, h.reshape(N, 1, D), w_projT.reshape(1, 1, D))
    return h_ordered.reshape(N, D), perm3.reshape(N)


NW = 32
CH = NPAD // NW
SCB = 64
NSB = CH // SCB


def _sc_reorder_body(h_hbm, sc_hbm, rank3_hbm, io_hbm, wp_hbm,
                     out_hbm, perm_hbm,
                     rows_v, rank2_v, io_v, sc_v, wp_v, sem):
    wid = lax.axis_index("s") * 2 + lax.axis_index("c")
    base = wid * CH
    pltpu.sync_copy(h_hbm.at[pl.ds(base, CH)], rows_v)
    pltpu.sync_copy(rank3_hbm.at[wid], rank2_v)
    pltpu.sync_copy(io_hbm.at[wid], io_v)
    pltpu.sync_copy(sc_hbm.at[pl.ds(base, CH)], sc_v)
    pltpu.sync_copy(wp_hbm, wp_v)

    def group_body(k, _):
        s16 = sc_v[pl.ds(k * 16, 16)]
        for j in range(16):
            r = k * 16 + j
            s = s16[j]
            for dk in range(D // 16):
                sl = pl.ds(dk * 16, 16)
                rows_v[r, sl] = rows_v[r, sl] + s * wp_v[sl]
        return _

    lax.fori_loop(0, CH // 16, group_body, 0)

    for j in range(NSB):
        pltpu.async_copy(rows_v.at[pl.ds(j * SCB, SCB)],
                         out_hbm.at[rank2_v.at[j]], sem).wait()
    for j in range(NSB):
        pltpu.async_copy(io_v.at[j], perm_hbm.at[rank2_v.at[j]], sem).wait()


def _sc_reorder(h, scores, rank, w_proj1d):
    hp = jnp.concatenate([h, jnp.zeros((NPAD - N, D), jnp.float32)])
    sp = jnp.concatenate([scores, jnp.zeros((NPAD - N,), jnp.float32)])
    kfn = pl.kernel(
        _sc_reorder_body,
        mesh=plsc.VectorSubcoreMesh(core_axis_name="c", subcore_axis_name="s"),
        compiler_params=pltpu.CompilerParams(use_tc_tiling_on_sc=False),
        out_type=[jax.ShapeDtypeStruct((NPAD, D), jnp.float32),
                  jax.ShapeDtypeStruct((NPAD,), jnp.int32)],
        scratch_types=[
            pltpu.VMEM((CH, D), jnp.float32),
            pltpu.VMEM((NSB, SCB), jnp.int32),
            pltpu.VMEM((NSB, SCB), jnp.int32),
            pltpu.VMEM((CH,), jnp.float32),
            pltpu.VMEM((D,), jnp.float32),
            pltpu.SemaphoreType.DMA,
        ],
    )
    iov = jnp.arange(NPAD, dtype=jnp.int32).reshape(NW, NSB, SCB)
    out, perm = kfn(hp, sp, rank.reshape(NW, NSB, SCB), iov, w_proj1d)
    return out[:N], perm[:N]


def kernel(h, edge_index, W, att_src, att_dst, bias, w_proj):
    x = _matvec(h, W)

    a_s = (x * att_src).sum(-1)
    a_d = (x * att_dst).sum(-1)
    loop = jnp.arange(N, dtype=edge_index.dtype)
    src = jnp.concatenate([edge_index[0], loop])
    dst = jnp.concatenate([edge_index[1], loop])
    ET = E + N
    iota_e = jnp.arange(ET, dtype=jnp.int32)
    ds, pi = jax.lax.sort((dst, iota_e), num_keys=1, is_stable=False)
    e = a_s[src] + a_d[dst]
    e = jnp.where(e > 0, e, 0.2 * e)
    m = jax.ops.segment_max(e[pi], ds, num_segments=N, indices_are_sorted=True)
    exs = jnp.exp(e[pi] - m[ds])
    den = jax.ops.segment_sum(exs, ds, num_segments=N, indices_are_sorted=True)
    alpha_s = exs / den[ds]
    val_s = alpha_s * x[:, 0][src][pi]
    out = jax.ops.segment_sum(val_s, ds, num_segments=N, indices_are_sorted=True)
    scores = out + bias[0]

    c = -scores
    c = jnp.where(c == 0.0, jnp.float32(0.0), c)
    b = jax.lax.bitcast_convert_type(c, jnp.int32)
    v = jnp.where(b >= 0, b, (~b) ^ jnp.int32(-2147483648))
    vpad = jnp.concatenate([v, jnp.full((NPAD - N,), jnp.int32(2147483647))])
    rank = _ranks(vpad)

    h_ordered, perm_idx = _sc_reorder(h, scores, rank, w_proj[:, 0])
    return (h_ordered, perm_idx, scores)


# speedup vs baseline: 1.8505x; 1.0003x over previous
import jax
import jax.numpy as jnp
from jax import lax
from jax.experimental import pallas as pl
from jax.experimental.pallas import tpu as pltpu
from jax.experimental.pallas import tpu_sc as plsc

N = 10000
E = 160000
D = 256
NPAD = 10240
RB = 1024


def _matvec_body(h_ref, w_ref, o_ref):
    o_ref[...] = jnp.dot(h_ref[...], w_ref[...], preferred_element_type=jnp.float32)


def _matvec(h, W):
    return pl.pallas_call(
        _matvec_body,
        grid=(10,),
        in_specs=[pl.BlockSpec((1000, D), lambda i: (i, 0)),
                  pl.BlockSpec((D, 1), lambda i: (0, 0))],
        out_specs=pl.BlockSpec((1000, 1), lambda i: (i, 0)),
        out_shape=jax.ShapeDtypeStruct((N, 1), jnp.float32),
    )(h, W)


def _rank_body(vi_ref, vj_ref, o_ref):
    i = pl.program_id(0)
    j = pl.program_id(1)
    a = vi_ref[...]
    b = vj_ref[...]
    A = a[:, None]
    B = b[None, :]
    gi = i * RB + jax.lax.broadcasted_iota(jnp.int32, (RB, RB), 0)
    gj = j * RB + jax.lax.broadcasted_iota(jnp.int32, (RB, RB), 1)
    less = (B < A) | ((B == A) & (gj < gi))
    cnt = jnp.sum(less.astype(jnp.int32), axis=1)

    @pl.when(j == 0)
    def _init():
        o_ref[...] = cnt

    @pl.when(j != 0)
    def _acc():
        o_ref[...] = o_ref[...] + cnt


def _ranks(v):
    return pl.pallas_call(
        _rank_body,
        grid=(NPAD // RB, NPAD // RB),
        in_specs=[pl.BlockSpec((RB,), lambda i, j: (i,)),
                  pl.BlockSpec((RB,), lambda i, j: (j,))],
        out_specs=pl.BlockSpec((RB,), lambda i, j: (i,)),
        out_shape=jax.ShapeDtypeStruct((NPAD,), jnp.int32),
        compiler_params=pltpu.CompilerParams(
            dimension_semantics=("arbitrary", "arbitrary")),
    )(v, v)


NW = 32
CH = NPAD // NW
SCB = 64
NSB = CH // SCB


def _sc_reorder_body(h_hbm, sc_hbm, rank3_hbm, io_hbm, wp_hbm,
                     out_hbm, perm_hbm,
                     rows_v, rank2_v, io_v, sc_v, wp_v, sem):
    wid = lax.axis_index("s") * 2 + lax.axis_index("c")
    base = wid * CH
    pltpu.sync_copy(h_hbm.at[pl.ds(base, CH)], rows_v)
    pltpu.sync_copy(rank3_hbm.at[wid], rank2_v)
    pltpu.sync_copy(io_hbm.at[wid], io_v)
    pltpu.sync_copy(sc_hbm.at[pl.ds(base, CH)], sc_v)
    pltpu.sync_copy(wp_hbm, wp_v)

    def group_body(k, _):
        s16 = sc_v[pl.ds(k * 16, 16)]
        for j in range(16):
            r = k * 16 + j
            s = s16[j]
            for dk in range(D // 16):
                sl = pl.ds(dk * 16, 16)
                rows_v[r, sl] = rows_v[r, sl] + s * wp_v[sl]
        return _

    lax.fori_loop(0, CH // 16, group_body, 0)

    for j in range(NSB):
        pltpu.async_copy(rows_v.at[pl.ds(j * SCB, SCB)],
                         out_hbm.at[rank2_v.at[j]], sem).wait()
    for j in range(NSB):
        pltpu.async_copy(io_v.at[j], perm_hbm.at[rank2_v.at[j]], sem).wait()


def _sc_reorder(h, scores, rank, w_proj1d):
    hp = jnp.concatenate([h, jnp.zeros((NPAD - N, D), jnp.float32)])
    sp = jnp.concatenate([scores, jnp.zeros((NPAD - N,), jnp.float32)])
    kfn = pl.kernel(
        _sc_reorder_body,
        mesh=plsc.VectorSubcoreMesh(core_axis_name="c", subcore_axis_name="s"),
        compiler_params=pltpu.CompilerParams(use_tc_tiling_on_sc=False),
        out_type=[jax.ShapeDtypeStruct((NPAD, D), jnp.float32),
                  jax.ShapeDtypeStruct((NPAD,), jnp.int32)],
        scratch_types=[
            pltpu.VMEM((CH, D), jnp.float32),
            pltpu.VMEM((NSB, SCB), jnp.int32),
            pltpu.VMEM((NSB, SCB), jnp.int32),
            pltpu.VMEM((CH,), jnp.float32),
            pltpu.VMEM((D,), jnp.float32),
            pltpu.SemaphoreType.DMA,
        ],
    )
    iov = jnp.arange(NPAD, dtype=jnp.int32).reshape(NW, NSB, SCB)
    out, perm = kfn(hp, sp, rank.reshape(NW, NSB, SCB), iov, w_proj1d)
    return out[:N], perm[:N]


def kernel(h, edge_index, W, att_src, att_dst, bias, w_proj):
    x = _matvec(h, W)

    a_s = (x * att_src).sum(-1)
    a_d = (x * att_dst).sum(-1)
    loop = jnp.arange(N, dtype=edge_index.dtype)
    src = jnp.concatenate([edge_index[0], loop])
    dst = jnp.concatenate([edge_index[1], loop])
    ET = E + N
    iota_e = jnp.arange(ET, dtype=jnp.int32)
    ds, pi = jax.lax.sort((dst, iota_e), num_keys=1, is_stable=False)
    e = a_s[src] + a_d[dst]
    e = jnp.where(e > 0, e, 0.2 * e)
    m = jax.ops.segment_max(e[pi], ds, num_segments=N, indices_are_sorted=True)
    exs = jnp.exp(e[pi] - m[ds])
    den = jax.ops.segment_sum(exs, ds, num_segments=N, indices_are_sorted=True)
    alpha_s = exs / den[ds]
    val_s = alpha_s * x[:, 0][src][pi]
    out = jax.ops.segment_sum(val_s, ds, num_segments=N, indices_are_sorted=True)
    scores = out + bias[0]

    c = -scores
    c = jnp.where(c == 0.0, jnp.float32(0.0), c)
    b = jax.lax.bitcast_convert_type(c, jnp.int32)
    v = jnp.where(b >= 0, b, (~b) ^ jnp.int32(-2147483648))
    vpad = jnp.concatenate([v, jnp.full((NPAD - N,), jnp.int32(2147483647))])
    rank = _ranks(vpad)

    h_ordered, perm_idx = _sc_reorder(h, scores, rank, w_proj[:, 0])
    return (h_ordered, perm_idx, scores)
